# baseline (speedup 1.0000x reference)
import functools

import jax
import jax.numpy as jnp
from jax import lax
from jax.experimental import pallas as pl
from jax.experimental.pallas import tpu as pltpu
from jax.experimental.pallas import tpu_sc as plsc

NC = 2
NS = 16
NW = NC * NS
CHUNK = 128
KG = 4


@functools.partial(jax.jit, static_argnums=(2,))
def _gather_rows(idx2d, table2, chunks_per_w):
    mesh = plsc.VectorSubcoreMesh(core_axis_name="c", subcore_axis_name="s")
    total = idx2d.shape[0] * CHUNK
    d2 = table2.shape[1]
    group = KG * CHUNK
    nsteps = chunks_per_w // KG

    @functools.partial(
        pl.kernel,
        out_type=jax.ShapeDtypeStruct((total, d2), jnp.uint16),
        mesh=mesh,
        scratch_types=[
            pltpu.VMEM((chunks_per_w, CHUNK), jnp.int32),
            pltpu.VMEM((group, d2), jnp.uint16),
            pltpu.VMEM((group, d2), jnp.uint16),
            pltpu.SemaphoreType.DMA,
            pltpu.SemaphoreType.DMA,
            pltpu.SemaphoreType.DMA,
            pltpu.SemaphoreType.DMA,
        ],
        compiler_params=pltpu.CompilerParams(use_tc_tiling_on_sc=False),
    )
    def run(tab_hbm, idx_hbm, out_hbm, idx_v, buf0, buf1, g0, g1, o0, o1):
        wid = lax.axis_index("s") * NC + lax.axis_index("c")
        chunk0 = wid * chunks_per_w
        row0 = chunk0 * CHUNK
        pltpu.sync_copy(idx_hbm.at[pl.ds(chunk0, chunks_per_w)], idx_v)

        bufs = (buf0, buf1)
        gsems = (g0, g1)
        osems = (o0, o1)

        def fire_g(s, b):
            for t in range(KG):
                pltpu.async_copy(
                    tab_hbm.at[idx_v.at[s * KG + t]],
                    bufs[b].at[pl.ds(t * CHUNK, CHUNK)],
                    gsems[b],
                )

        def drain_g(b):
            pltpu.make_async_copy(
                tab_hbm.at[pl.ds(0, group)], bufs[b], gsems[b]
            ).wait()

        def fire_o(s, b):
            pltpu.async_copy(
                bufs[b], out_hbm.at[pl.ds(row0 + s * group, group)], osems[b]
            )

        def wait_o(b):
            pltpu.make_async_copy(
                bufs[b], out_hbm.at[pl.ds(0, group)], osems[b]
            ).wait()

        fire_g(0, 0)
        fire_g(1, 1)
        drain_g(0)
        fire_o(0, 0)

        def body(i, carry):
            drain_g(1)
            fire_o(2 * i + 1, 1)
            wait_o(0)
            fire_g(2 * i + 2, 0)
            drain_g(0)
            fire_o(2 * i + 2, 0)
            wait_o(1)
            fire_g(2 * i + 3, 1)
            return carry

        lax.fori_loop(0, (nsteps - 2) // 2, body, 0)

        drain_g(1)
        wait_o(0)
        fire_o(nsteps - 1, 1)
        wait_o(1)

    return run(table2, idx2d)


def kernel(indices, table):
    b, h = indices.shape
    v, d = table.shape
    total = b * h
    assert total % (NW * CHUNK * KG) == 0
    chunks_per_w = total // (NW * CHUNK)
    idx2d = indices.reshape(total // CHUNK, CHUNK).astype(jnp.int32)
    table2 = lax.bitcast_convert_type(table, jnp.uint16).reshape(v, 2 * d)
    out2 = _gather_rows(idx2d, table2, chunks_per_w)
    return lax.bitcast_convert_type(
        out2.reshape(b, h, d, 2), jnp.float32
    )


# speedup vs baseline: 6.6194x; 6.6194x over previous
import functools

import jax
import jax.numpy as jnp
from jax import lax
from jax.experimental import pallas as pl
from jax.experimental.pallas import tpu as pltpu
from jax.experimental.pallas import tpu_sc as plsc

NC = 2
NS = 16
NW = NC * NS
CHUNK = 128
KG = 4


@functools.partial(jax.jit, static_argnums=(2, 3))
def _gather_rows(idx2d, table, chunks_per_w, d):
    mesh = plsc.VectorSubcoreMesh(core_axis_name="c", subcore_axis_name="s")
    total = idx2d.shape[0] * CHUNK
    group = KG * CHUNK
    nsteps = chunks_per_w // KG

    @functools.partial(
        pl.kernel,
        out_type=jax.ShapeDtypeStruct((total, d), jnp.float32),
        mesh=mesh,
        scratch_types=[
            pltpu.VMEM((chunks_per_w, CHUNK), jnp.int32),
            pltpu.VMEM((group, d), jnp.float32),
            pltpu.VMEM((group, d), jnp.float32),
            pltpu.SemaphoreType.DMA,
            pltpu.SemaphoreType.DMA,
            pltpu.SemaphoreType.DMA,
            pltpu.SemaphoreType.DMA,
        ],
        compiler_params=pltpu.CompilerParams(use_tc_tiling_on_sc=False),
    )
    def run(tab_hbm, idx_hbm, out_hbm, idx_v, buf0, buf1, g0, g1, o0, o1):
        wid = lax.axis_index("s") * NC + lax.axis_index("c")
        chunk0 = wid * chunks_per_w
        row0 = chunk0 * CHUNK
        pltpu.sync_copy(idx_hbm.at[pl.ds(chunk0, chunks_per_w)], idx_v)

        bufs = (buf0, buf1)
        gsems = (g0, g1)
        osems = (o0, o1)

        def fire_g(s, b):
            for t in range(KG):
                pltpu.async_copy(
                    tab_hbm.at[idx_v.at[s * KG + t]],
                    bufs[b].at[pl.ds(t * CHUNK, CHUNK)],
                    gsems[b],
                )

        def drain_g(b):
            pltpu.make_async_copy(
                tab_hbm.at[pl.ds(0, group)], bufs[b], gsems[b]
            ).wait()

        def fire_o(s, b):
            pltpu.async_copy(
                bufs[b], out_hbm.at[pl.ds(row0 + s * group, group)], osems[b]
            )

        def wait_o(b):
            pltpu.make_async_copy(
                bufs[b], out_hbm.at[pl.ds(0, group)], osems[b]
            ).wait()

        fire_g(0, 0)
        fire_g(1, 1)
        drain_g(0)
        fire_o(0, 0)

        def body(i, carry):
            drain_g(1)
            fire_o(2 * i + 1, 1)
            wait_o(0)
            fire_g(2 * i + 2, 0)
            drain_g(0)
            fire_o(2 * i + 2, 0)
            wait_o(1)
            fire_g(2 * i + 3, 1)
            return carry

        lax.fori_loop(0, (nsteps - 2) // 2, body, 0)

        drain_g(1)
        wait_o(0)
        fire_o(nsteps - 1, 1)
        wait_o(1)

    return run(table, idx2d)


def kernel(indices, table):
    b, h = indices.shape
    v, d = table.shape
    total = b * h
    assert total % (NW * CHUNK * KG) == 0
    chunks_per_w = total // (NW * CHUNK)
    idx2d = indices.reshape(total // CHUNK, CHUNK).astype(jnp.int32)
    out = _gather_rows(idx2d, table, chunks_per_w, d)
    return out.reshape(b, h, d)


# speedup vs baseline: 6.6293x; 1.0015x over previous
import functools

import jax
import jax.numpy as jnp
from jax import lax
from jax.experimental import pallas as pl
from jax.experimental.pallas import tpu as pltpu
from jax.experimental.pallas import tpu_sc as plsc

NC = 2
NS = 16
NW = NC * NS
CHUNK = 128
KG = 5


@functools.partial(jax.jit, static_argnums=(2, 3))
def _gather_rows(idx2d, table, chunks_per_w, d):
    mesh = plsc.VectorSubcoreMesh(core_axis_name="c", subcore_axis_name="s")
    total = idx2d.shape[0] * CHUNK
    group = KG * CHUNK
    nsteps = chunks_per_w // KG

    @functools.partial(
        pl.kernel,
        out_type=jax.ShapeDtypeStruct((total, d), jnp.float32),
        mesh=mesh,
        scratch_types=[
            pltpu.VMEM((chunks_per_w, CHUNK), jnp.int32),
            pltpu.VMEM((group, d), jnp.float32),
            pltpu.VMEM((group, d), jnp.float32),
            pltpu.SemaphoreType.DMA,
            pltpu.SemaphoreType.DMA,
            pltpu.SemaphoreType.DMA,
            pltpu.SemaphoreType.DMA,
        ],
        compiler_params=pltpu.CompilerParams(use_tc_tiling_on_sc=False),
    )
    def run(tab_hbm, idx_hbm, out_hbm, idx_v, buf0, buf1, g0, g1, o0, o1):
        wid = lax.axis_index("s") * NC + lax.axis_index("c")
        chunk0 = wid * chunks_per_w
        row0 = chunk0 * CHUNK
        pltpu.sync_copy(idx_hbm.at[pl.ds(chunk0, chunks_per_w)], idx_v)

        bufs = (buf0, buf1)
        gsems = (g0, g1)
        osems = (o0, o1)

        def fire_g(s, b):
            for t in range(KG):
                pltpu.async_copy(
                    tab_hbm.at[idx_v.at[s * KG + t]],
                    bufs[b].at[pl.ds(t * CHUNK, CHUNK)],
                    gsems[b],
                )

        def drain_g(b):
            pltpu.make_async_copy(
                tab_hbm.at[pl.ds(0, group)], bufs[b], gsems[b]
            ).wait()

        def fire_o(s, b):
            pltpu.async_copy(
                bufs[b], out_hbm.at[pl.ds(row0 + s * group, group)], osems[b]
            )

        def wait_o(b):
            pltpu.make_async_copy(
                bufs[b], out_hbm.at[pl.ds(0, group)], osems[b]
            ).wait()

        fire_g(0, 0)
        fire_g(1, 1)
        drain_g(0)
        fire_o(0, 0)

        def body(i, carry):
            drain_g(1)
            fire_o(2 * i + 1, 1)
            wait_o(0)
            fire_g(2 * i + 2, 0)
            drain_g(0)
            fire_o(2 * i + 2, 0)
            wait_o(1)
            fire_g(2 * i + 3, 1)
            return carry

        lax.fori_loop(0, (nsteps - 2) // 2, body, 0)

        drain_g(1)
        wait_o(0)
        fire_o(nsteps - 1, 1)
        wait_o(1)

    return run(table, idx2d)


def kernel(indices, table):
    b, h = indices.shape
    v, d = table.shape
    total = b * h
    assert total % (NW * CHUNK * KG) == 0
    chunks_per_w = total // (NW * CHUNK)
    idx2d = indices.reshape(total // CHUNK, CHUNK).astype(jnp.int32)
    out = _gather_rows(idx2d, table, chunks_per_w, d)
    return out.reshape(b, h, d)
